# baseline (speedup 1.0000x reference)
import functools

import jax
import jax.numpy as jnp
from jax import lax
from jax.experimental import pallas as pl
from jax.experimental.pallas import tpu as pltpu

_K_STEPS = 10
_LAMBDA_AMP = 0.5


def _fused_kernel(xT_ref, adj_ref, w1T_ref, b1_ref, w2T_ref, b2_ref,
                  out_ref, adj_bf_ref, hh_ref, xa_ref, xb_ref, ax_ref, *,
                  n_steps, n_tiles, tile_n, half_n, lam):
    p = pl.program_id(0)
    j = pl.program_id(1)
    col = pl.multiple_of(j * tile_n, tile_n)

    @pl.when(p == 0)
    def _encode_and_stash_top():
        adj_bf_ref[0:half_n, pl.ds(col, tile_n)] = (
            adj_ref[...].astype(jnp.bfloat16))
        h = jnp.dot(w1T_ref[...], xT_ref[...],
                    preferred_element_type=jnp.float32)
        h = jnp.maximum(h + b1_ref[...], 0.0)
        hh = jnp.dot(w2T_ref[...], h.astype(jnp.bfloat16),
                     preferred_element_type=jnp.float32) + b2_ref[...]
        hh_ref[:, pl.ds(col, tile_n)] = hh

        @pl.when(j == n_tiles - 1)
        def _():
            xa_ref[...] = hh_ref[...].astype(jnp.bfloat16)

    def _dot(src_ref, t):
        cc = pl.multiple_of(t * tile_n, tile_n)
        ax_ref[:, pl.ds(cc, tile_n)] = jnp.dot(
            src_ref[...], adj_bf_ref[:, pl.ds(cc, tile_n)],
            preferred_element_type=jnp.float32)

    def _prox_math(t):
        cc = pl.multiple_of(t * tile_n, tile_n)
        ax = ax_ref[:, pl.ds(cc, tile_n)]
        hh = hh_ref[:, pl.ds(cc, tile_n)]
        d = ax - hh
        rn = jnp.sqrt(jnp.sum(d * d, axis=0, keepdims=True))
        scale = jnp.where(rn > lam, (rn - lam) / jnp.maximum(rn, 1e-30), 0.0)
        return cc, hh + scale * d

    def _prox(dst_ref, t):
        cc, xn = _prox_math(t)
        dst_ref[:, pl.ds(cc, tile_n)] = xn.astype(jnp.bfloat16)

    def _prox_final(t):
        cc, xn = _prox_math(t)
        m = jnp.max(xn, axis=0, keepdims=True)
        sh = xn - m
        lse = jnp.log(jnp.sum(jnp.exp(sh), axis=0, keepdims=True))
        out_ref[:, pl.ds(cc, tile_n)] = sh - lse

    @pl.when(p == 1)
    def _stash_bottom_and_step1():
        adj_bf_ref[half_n:2 * half_n, pl.ds(col, tile_n)] = (
            adj_ref[...].astype(jnp.bfloat16))
        _dot(xa_ref, j)
        _prox(xb_ref, j)

    for parity, src_ref, dst_ref in ((0, xa_ref, xb_ref), (1, xb_ref, xa_ref)):
        is_amp_body = jnp.logical_and(p >= 2, jnp.logical_and(
            p % 2 == 1 - parity, j == 0))

        @pl.when(is_amp_body)
        def _amp_step(src_ref=src_ref, dst_ref=dst_ref):
            _dot(src_ref, 0)

            def _loop(t, carry):
                @pl.when(p < n_steps)
                def _():
                    _prox(dst_ref, t - 1)

                @pl.when(p == n_steps)
                def _():
                    _prox_final(t - 1)

                _dot(src_ref, t)
                return carry

            lax.fori_loop(1, n_tiles, _loop, 0, unroll=False)

            @pl.when(p < n_steps)
            def _():
                _prox(dst_ref, n_tiles - 1)

            @pl.when(p == n_steps)
            def _():
                _prox_final(n_tiles - 1)


def kernel(x, adj, w1, b1, w2, b2):
    N, F = x.shape
    H = w1.shape[1]
    C = w2.shape[1]
    assert adj.shape == (N, N)
    assert C == 128 and N % 1024 == 0, (C, N)

    tn = 512
    n_tiles = N // tn
    f32 = jnp.float32
    bf16 = jnp.bfloat16

    gamma = 1.0 / (2.0 * (1.0 - _LAMBDA_AMP))
    lam = float(gamma * _LAMBDA_AMP)

    xT = x.T.astype(bf16)
    w1T = w1.T.astype(bf16)
    b1c = b1.astype(f32).reshape(H, 1)
    w2T = w2.T.astype(bf16)
    b2c = b2.astype(f32).reshape(C, 1)

    cost = pl.CostEstimate(
        flops=int(2 * N * F * H + 2 * N * H * C
                  + 2 * _K_STEPS * N * N * C + 12 * _K_STEPS * N * C),
        transcendentals=int(2 * _K_STEPS * N + C * N),
        bytes_accessed=int(4 * N * N + 2 * F * N + 4 * 2 * C * N),
    )

    half_n = N // 2
    body = functools.partial(_fused_kernel, n_steps=_K_STEPS,
                             n_tiles=n_tiles, tile_n=tn, half_n=half_n,
                             lam=lam)

    outT = pl.pallas_call(
        body,
        out_shape=jax.ShapeDtypeStruct((C, N), f32),
        grid_spec=pltpu.PrefetchScalarGridSpec(
            num_scalar_prefetch=0,
            grid=(_K_STEPS + 1, n_tiles),
            in_specs=[
                pl.BlockSpec((F, tn), lambda p, j: (0, jnp.where(p == 0, j, 0))),
                pl.BlockSpec((half_n, tn),
                             lambda p, j: (jnp.where(p < 2, p, 0),
                                           jnp.where(p < 2, j, 0))),
                pl.BlockSpec((H, F), lambda p, j: (0, 0)),
                pl.BlockSpec((H, 1), lambda p, j: (0, 0)),
                pl.BlockSpec((C, H), lambda p, j: (0, 0)),
                pl.BlockSpec((C, 1), lambda p, j: (0, 0)),
            ],
            out_specs=pl.BlockSpec((C, N), lambda p, j: (0, 0)),
            scratch_shapes=[
                pltpu.VMEM((N, N), bf16),
                pltpu.VMEM((C, N), f32),
                pltpu.VMEM((C, N), bf16),
                pltpu.VMEM((C, N), bf16),
                pltpu.VMEM((C, N), f32),
            ],
        ),
        compiler_params=pltpu.CompilerParams(
            dimension_semantics=("arbitrary", "arbitrary"),
            vmem_limit_bytes=56 * 1024 * 1024,
        ),
        cost_estimate=cost,
    )(xT, adj, w1T, b1c, w2T, b2c)

    return outT.T


# speedup vs baseline: 1.0562x; 1.0562x over previous
import functools

import jax
import jax.numpy as jnp
from jax import lax
from jax.experimental import pallas as pl
from jax.experimental.pallas import tpu as pltpu

_K_STEPS = 10
_LAMBDA_AMP = 0.5


def _fused_kernel(xT_ref, adj_ref, w1T_ref, b1_ref, w2T_ref, b2_ref,
                  out_ref, adj_bf_ref, hh_ref, xa_ref, xb_ref, ax_ref, *,
                  n_steps, n_tiles, tile_n, half_n, lam):
    g = pl.program_id(0)
    last_g = 2 * n_tiles + n_steps - 2

    def _dot(src_ref, t):
        cc = pl.multiple_of(t * tile_n, tile_n)
        ax_ref[:, pl.ds(cc, tile_n)] = jnp.dot(
            src_ref[...], adj_bf_ref[:, pl.ds(cc, tile_n)],
            preferred_element_type=jnp.float32)

    def _prox_math(t):
        cc = pl.multiple_of(t * tile_n, tile_n)
        ax = ax_ref[:, pl.ds(cc, tile_n)]
        hh = hh_ref[:, pl.ds(cc, tile_n)]
        d = ax - hh
        s2 = jnp.sum(d * d, axis=0, keepdims=True)
        scale = jnp.maximum(1.0 - lam * jax.lax.rsqrt(jnp.maximum(s2, 1e-30)),
                            0.0)
        return cc, hh + scale * d

    def _prox(dst_ref, t):
        cc, xn = _prox_math(t)
        dst_ref[:, pl.ds(cc, tile_n)] = xn.astype(jnp.bfloat16)

    def _prox_final(t):
        cc, xn = _prox_math(t)
        m = jnp.max(xn, axis=0, keepdims=True)
        sh = xn - m
        lse = jnp.log(jnp.sum(jnp.exp(sh), axis=0, keepdims=True))
        out_ref[:, pl.ds(cc, tile_n)] = sh - lse

    @pl.when(g < n_tiles)
    def _encode_and_stash_top():
        col = pl.multiple_of(g * tile_n, tile_n)
        adj_bf_ref[0:half_n, pl.ds(col, tile_n)] = (
            adj_ref[...].astype(jnp.bfloat16))
        h = jnp.dot(w1T_ref[...], xT_ref[...],
                    preferred_element_type=jnp.float32)
        h = jnp.maximum(h + b1_ref[...], 0.0)
        hh = jnp.dot(w2T_ref[...], h.astype(jnp.bfloat16),
                     preferred_element_type=jnp.float32) + b2_ref[...]
        hh_ref[:, pl.ds(col, tile_n)] = hh

        @pl.when(g == n_tiles - 1)
        def _():
            xa_ref[...] = hh_ref[...].astype(jnp.bfloat16)

    @pl.when(jnp.logical_and(g >= n_tiles, g < 2 * n_tiles))
    def _stash_bottom_and_step1():
        t = g - n_tiles
        col = pl.multiple_of(t * tile_n, tile_n)
        adj_bf_ref[half_n:2 * half_n, pl.ds(col, tile_n)] = (
            adj_ref[...].astype(jnp.bfloat16))
        _dot(xa_ref, t)
        _prox(xb_ref, t)

    for parity, src_ref, dst_ref in ((0, xa_ref, xb_ref), (1, xb_ref, xa_ref)):
        is_amp_body = jnp.logical_and(g >= 2 * n_tiles, g % 2 == 1 - parity)

        @pl.when(is_amp_body)
        def _amp_step(src_ref=src_ref, dst_ref=dst_ref):
            _dot(src_ref, 0)

            def _loop(t, carry):
                @pl.when(g < last_g)
                def _():
                    _prox(dst_ref, t - 1)

                @pl.when(g == last_g)
                def _():
                    _prox_final(t - 1)

                _dot(src_ref, t)
                return carry

            lax.fori_loop(1, n_tiles, _loop, 0, unroll=False)

            @pl.when(g < last_g)
            def _():
                _prox(dst_ref, n_tiles - 1)

            @pl.when(g == last_g)
            def _():
                _prox_final(n_tiles - 1)


def kernel(x, adj, w1, b1, w2, b2):
    N, F = x.shape
    H = w1.shape[1]
    C = w2.shape[1]
    assert adj.shape == (N, N)
    assert C == 128 and N % 1024 == 0, (C, N)

    tn = 512
    n_tiles = N // tn
    f32 = jnp.float32
    bf16 = jnp.bfloat16

    gamma = 1.0 / (2.0 * (1.0 - _LAMBDA_AMP))
    lam = float(gamma * _LAMBDA_AMP)

    xT = x.T.astype(bf16)
    w1T = w1.T.astype(bf16)
    b1c = b1.astype(f32).reshape(H, 1)
    w2T = w2.T.astype(bf16)
    b2c = b2.astype(f32).reshape(C, 1)

    cost = pl.CostEstimate(
        flops=int(2 * N * F * H + 2 * N * H * C
                  + 2 * _K_STEPS * N * N * C + 12 * _K_STEPS * N * C),
        transcendentals=int(2 * _K_STEPS * N + C * N),
        bytes_accessed=int(4 * N * N + 2 * F * N + 4 * 2 * C * N),
    )

    half_n = N // 2
    nt = n_tiles
    body = functools.partial(_fused_kernel, n_steps=_K_STEPS,
                             n_tiles=n_tiles, tile_n=tn, half_n=half_n,
                             lam=lam)

    outT = pl.pallas_call(
        body,
        out_shape=jax.ShapeDtypeStruct((C, N), f32),
        grid_spec=pltpu.PrefetchScalarGridSpec(
            num_scalar_prefetch=0,
            grid=(2 * n_tiles + _K_STEPS - 1,),
            in_specs=[
                pl.BlockSpec((F, tn),
                             lambda g: (0, jnp.where(g < nt, g, 0))),
                pl.BlockSpec((half_n, tn),
                             lambda g: (jnp.where(g < nt, 0,
                                                  jnp.where(g < 2 * nt, 1, 0)),
                                        jnp.where(g < 2 * nt,
                                                  lax.rem(g, nt), 0))),
                pl.BlockSpec((H, F), lambda g: (0, 0)),
                pl.BlockSpec((H, 1), lambda g: (0, 0)),
                pl.BlockSpec((C, H), lambda g: (0, 0)),
                pl.BlockSpec((C, 1), lambda g: (0, 0)),
            ],
            out_specs=pl.BlockSpec((C, N), lambda g: (0, 0)),
            scratch_shapes=[
                pltpu.VMEM((N, N), bf16),
                pltpu.VMEM((C, N), f32),
                pltpu.VMEM((C, N), bf16),
                pltpu.VMEM((C, N), bf16),
                pltpu.VMEM((C, N), f32),
            ],
        ),
        compiler_params=pltpu.CompilerParams(
            dimension_semantics=("arbitrary",),
            vmem_limit_bytes=56 * 1024 * 1024,
        ),
        cost_estimate=cost,
    )(xT, adj, w1T, b1c, w2T, b2c)

    return outT.T


# speedup vs baseline: 1.1099x; 1.0508x over previous
import functools

import jax
import jax.numpy as jnp
from jax import lax
from jax.experimental import pallas as pl
from jax.experimental.pallas import tpu as pltpu

_K_STEPS = 10
_LAMBDA_AMP = 0.5


def _fused_kernel(xT_ref, adj_ref, w1T_ref, b1_ref, w2T_ref, b2_ref,
                  out_ref, adj_bf_ref, hh_ref, xa_ref, xb_ref, ax_ref, *,
                  n_steps, n_tiles, tile_n, half_n, lam):
    g = pl.program_id(0)
    last_g = 2 * n_tiles + n_steps - 2

    def _dot(src_ref, t):
        cc = pl.multiple_of(t * tile_n, tile_n)
        ax_ref[:, pl.ds(cc, tile_n)] = jnp.dot(
            src_ref[...], adj_bf_ref[:, pl.ds(cc, tile_n)],
            preferred_element_type=jnp.float32)

    def _prox_math(t):
        cc = pl.multiple_of(t * tile_n, tile_n)
        ax = ax_ref[:, pl.ds(cc, tile_n)]
        hh = hh_ref[:, pl.ds(cc, tile_n)]
        d = ax - hh
        s2 = jnp.sum(d * d, axis=0, keepdims=True)
        scale = jnp.maximum(1.0 - lam * jax.lax.rsqrt(jnp.maximum(s2, 1e-30)),
                            0.0)
        return cc, hh + scale * d

    def _prox(dst_ref, t):
        cc, xn = _prox_math(t)
        dst_ref[:, pl.ds(cc, tile_n)] = xn.astype(jnp.bfloat16)

    def _prox_final(t):
        cc, xn = _prox_math(t)
        m = jnp.max(xn, axis=0, keepdims=True)
        sh = xn - m
        lse = jnp.log(jnp.sum(jnp.exp(sh), axis=0, keepdims=True))
        out_ref[:, pl.ds(cc, tile_n)] = sh - lse

    @pl.when(g < n_tiles)
    def _encode_and_stash_top():
        col = pl.multiple_of(g * tile_n, tile_n)
        adj_bf_ref[0:half_n, pl.ds(col, tile_n)] = (
            adj_ref[...].astype(jnp.bfloat16))
        h = jnp.dot(w1T_ref[...], xT_ref[...],
                    preferred_element_type=jnp.float32)
        h = jnp.maximum(h + b1_ref[...], 0.0)
        hh = jnp.dot(w2T_ref[...], h.astype(jnp.bfloat16),
                     preferred_element_type=jnp.float32) + b2_ref[...]
        hh_ref[:, pl.ds(col, tile_n)] = hh

        @pl.when(g == n_tiles - 1)
        def _():
            xa_ref[...] = hh_ref[...].astype(jnp.bfloat16)

    @pl.when(jnp.logical_and(g >= n_tiles, g < 2 * n_tiles))
    def _stash_bottom_and_step1():
        t = g - n_tiles
        col = pl.multiple_of(t * tile_n, tile_n)
        adj_bf_ref[half_n:2 * half_n, pl.ds(col, tile_n)] = (
            adj_ref[...].astype(jnp.bfloat16))
        _dot(xa_ref, t)
        _prox(xb_ref, t)

    for parity, src_ref, dst_ref in ((0, xa_ref, xb_ref), (1, xb_ref, xa_ref)):
        is_amp_body = jnp.logical_and(
            jnp.logical_and(g >= 2 * n_tiles, g < last_g),
            g % 2 == 1 - parity)

        @pl.when(is_amp_body)
        def _amp_step(src_ref=src_ref, dst_ref=dst_ref):
            _dot(src_ref, 0)

            def _loop(t, carry, src_ref=src_ref, dst_ref=dst_ref):
                _prox(dst_ref, t - 1)
                _dot(src_ref, t)
                return carry

            lax.fori_loop(1, n_tiles, _loop, 0, unroll=False)
            _prox(dst_ref, n_tiles - 1)

    @pl.when(g == last_g)
    def _final_step():
        src_ref = xb_ref if _K_STEPS % 2 == 0 else xa_ref
        _dot(src_ref, 0)

        def _loop(t, carry, src_ref=src_ref):
            _prox_final(t - 1)
            _dot(src_ref, t)
            return carry

        lax.fori_loop(1, n_tiles, _loop, 0, unroll=False)
        _prox_final(n_tiles - 1)


def kernel(x, adj, w1, b1, w2, b2):
    N, F = x.shape
    H = w1.shape[1]
    C = w2.shape[1]
    assert adj.shape == (N, N)
    assert C == 128 and N % 1024 == 0, (C, N)

    tn = 512
    n_tiles = N // tn
    f32 = jnp.float32
    bf16 = jnp.bfloat16

    gamma = 1.0 / (2.0 * (1.0 - _LAMBDA_AMP))
    lam = float(gamma * _LAMBDA_AMP)

    xT = x.T.astype(bf16)
    w1T = w1.T.astype(bf16)
    b1c = b1.astype(f32).reshape(H, 1)
    w2T = w2.T.astype(bf16)
    b2c = b2.astype(f32).reshape(C, 1)

    cost = pl.CostEstimate(
        flops=int(2 * N * F * H + 2 * N * H * C
                  + 2 * _K_STEPS * N * N * C + 12 * _K_STEPS * N * C),
        transcendentals=int(2 * _K_STEPS * N + C * N),
        bytes_accessed=int(4 * N * N + 2 * F * N + 4 * 2 * C * N),
    )

    half_n = N // 2
    nt = n_tiles
    body = functools.partial(_fused_kernel, n_steps=_K_STEPS,
                             n_tiles=n_tiles, tile_n=tn, half_n=half_n,
                             lam=lam)

    outT = pl.pallas_call(
        body,
        out_shape=jax.ShapeDtypeStruct((C, N), f32),
        grid_spec=pltpu.PrefetchScalarGridSpec(
            num_scalar_prefetch=0,
            grid=(2 * n_tiles + _K_STEPS - 1,),
            in_specs=[
                pl.BlockSpec((F, tn),
                             lambda g: (0, jnp.where(g < nt, g, 0))),
                pl.BlockSpec((half_n, tn),
                             lambda g: (jnp.where(g < nt, 0,
                                                  jnp.where(g < 2 * nt, 1, 0)),
                                        jnp.where(g < 2 * nt,
                                                  lax.rem(g, nt), 0))),
                pl.BlockSpec((H, F), lambda g: (0, 0)),
                pl.BlockSpec((H, 1), lambda g: (0, 0)),
                pl.BlockSpec((C, H), lambda g: (0, 0)),
                pl.BlockSpec((C, 1), lambda g: (0, 0)),
            ],
            out_specs=pl.BlockSpec((C, N), lambda g: (0, 0)),
            scratch_shapes=[
                pltpu.VMEM((N, N), bf16),
                pltpu.VMEM((C, N), f32),
                pltpu.VMEM((C, N), bf16),
                pltpu.VMEM((C, N), bf16),
                pltpu.VMEM((C, N), f32),
            ],
        ),
        compiler_params=pltpu.CompilerParams(
            dimension_semantics=("arbitrary",),
            vmem_limit_bytes=56 * 1024 * 1024,
        ),
        cost_estimate=cost,
    )(xT, adj, w1T, b1c, w2T, b2c)

    return outT.T
